# baseline (speedup 1.0000x reference)
import functools

import jax
import jax.numpy as jnp
from jax import lax
from jax.experimental import pallas as pl
from jax.experimental.pallas import tpu as pltpu
from jax.experimental.pallas import tpu_sc as plsc


def _argmax_body(num_k_blocks, kb, x_ref, e_ref, idx_ref, rmax_ref, ridx_ref):
    k = pl.program_id(0)
    t = pl.program_id(1)
    tb = x_ref.shape[0]
    p = lax.dot_general(
        x_ref[...], e_ref[...], (((1,), (1,)), ((), ())),
        preferred_element_type=jnp.float32)
    lm = jnp.max(p, axis=1, keepdims=True)
    iota = lax.broadcasted_iota(jnp.int32, (tb, kb), 1)
    cand = jnp.where(p == lm, iota, kb * num_k_blocks)
    li = jnp.min(cand, axis=1, keepdims=True) + k * kb

    sl = pl.ds(t * tb, tb)

    @pl.when(k == 0)
    def _():
        rmax_ref[sl, :] = jnp.full((tb, 1), -jnp.inf, jnp.float32)
        ridx_ref[sl, :] = jnp.zeros((tb, 1), jnp.int32)

    better = lm > rmax_ref[sl, :]
    newm = jnp.where(better, lm, rmax_ref[sl, :])
    newi = jnp.where(better, li, ridx_ref[sl, :])
    rmax_ref[sl, :] = newm
    ridx_ref[sl, :] = newi

    @pl.when(k == num_k_blocks - 1)
    def _():
        idx_ref[...] = newi


def _distance_argmax(x2d, emb, tb=256, kb=1024, interpret=False):
    n, d = x2d.shape
    kk = emb.shape[0]
    n_t, n_k = n // tb, kk // kb
    return pl.pallas_call(
        functools.partial(_argmax_body, n_k, kb),
        grid=(n_k, n_t),
        in_specs=[
            pl.BlockSpec((tb, d), lambda k, t: (t, 0)),
            pl.BlockSpec((kb, d), lambda k, t: (k, 0)),
        ],
        out_specs=pl.BlockSpec((tb, 1), lambda k, t: (t, 0)),
        out_shape=jax.ShapeDtypeStruct((n, 1), jnp.int32),
        scratch_shapes=[
            pltpu.VMEM((n, 1), jnp.float32),
            pltpu.VMEM((n, 1), jnp.int32),
        ],
        interpret=interpret,
    )(x2d, emb)


def _sc_gather(emb, idx_flat, interpret=False):
    kk, d = emb.shape
    n = idx_flat.shape[0]
    info = plsc.get_sparse_core_info()
    nc, ns = info.num_cores, info.num_subcores
    nw = nc * ns
    bpw = n // nw
    ch = min(128, bpw)
    nch = bpw // ch
    idx2d = idx_flat.reshape(nw * nch, ch)
    mesh = plsc.VectorSubcoreMesh(core_axis_name="c", subcore_axis_name="s")

    @functools.partial(
        pl.kernel, mesh=mesh,
        out_type=jax.ShapeDtypeStruct((n, d), jnp.float32),
        scratch_types=[
            pltpu.VMEM((nch, ch), jnp.int32),
            pltpu.VMEM((bpw, d), jnp.float32),
            pltpu.SemaphoreType.DMA,
        ],
    )
    def gather_kernel(table_hbm, idx_hbm, out_hbm, idx_v, rows_v, sem):
        wid = lax.axis_index("s") * nc + lax.axis_index("c")
        pltpu.sync_copy(idx_hbm.at[pl.ds(wid * nch, nch)], idx_v)
        copies = [
            pltpu.async_copy(
                table_hbm.at[idx_v.at[j]], rows_v.at[pl.ds(j * ch, ch)], sem)
            for j in range(nch)
        ]
        for c in copies:
            c.wait()
        pltpu.sync_copy(rows_v, out_hbm.at[pl.ds(wid * bpw, bpw)])

    return gather_kernel(emb, idx2d)


def _loss_body(x_ref, e_ref, out_ref):
    i = pl.program_id(0)
    s = jnp.sum(jnp.abs(x_ref[...] - e_ref[...]))

    @pl.when(i == 0)
    def _():
        out_ref[0, 0] = 0.0

    out_ref[0, 0] += s


def _l1_sum(x2d, e2d, tb=512, interpret=False):
    n, d = x2d.shape
    return pl.pallas_call(
        _loss_body,
        grid=(n // tb,),
        in_specs=[
            pl.BlockSpec((tb, d), lambda i: (i, 0)),
            pl.BlockSpec((tb, d), lambda i: (i, 0)),
        ],
        out_specs=pl.BlockSpec(memory_space=pltpu.SMEM),
        out_shape=jax.ShapeDtypeStruct((1, 1), jnp.float32),
        interpret=interpret,
    )(x2d, e2d)


def kernel(x, embeddings):
    b, t, d = x.shape
    x2d = x.reshape(b * t, d)
    idx = _distance_argmax(x2d, embeddings)
    e2d = _sc_gather(embeddings, idx.reshape(b * t))
    total = _l1_sum(x2d, e2d)
    return total[0, 0] * (2.0 / x.size)
